# baseline (speedup 1.0000x reference)
import functools
import jax
import jax.numpy as jnp
from jax import lax
from jax.experimental import pallas as pl
from jax.experimental.pallas import tpu as pltpu
from jax.experimental.pallas import tpu_sc as plsc

B, T, D = 4, 4096, 2048
K = T // 2
ROWS, LANES = 32, 128
KROWS = K // LANES


_BT = 2048
_NSTEP = T // _BT


def _scores_kernel(x_ref, w_ref, o_ref):
    o_ref[0] = lax.dot_general(
        w_ref[...], x_ref[...], (((1,), (1,)), ((), ())),
        preferred_element_type=jnp.float32)


def _scores(x2d, w2d, b0, nb):
    return pl.pallas_call(
        _scores_kernel,
        grid=(nb * _NSTEP,),
        in_specs=[
            pl.BlockSpec((_BT, D), lambda i, b0=b0: (b0 * _NSTEP + i, 0)),
            pl.BlockSpec((1, D), lambda i: (0, 0)),
        ],
        out_specs=pl.BlockSpec((1, 1, _BT), lambda i: (i, 0, 0)),
        out_shape=jax.ShapeDtypeStruct((nb * _NSTEP, 1, _BT), jnp.float32),
    )(x2d, w2d)


def _topk_kernel(b0, s_ref, i_ref, f_ref):
    b = pl.program_id(0) + b0
    s2 = s_ref[0]
    rows = lax.broadcasted_iota(jnp.int32, (ROWS, LANES), 0)
    lanes = lax.broadcasted_iota(jnp.int32, (ROWS, LANES), 1)
    i2 = rows * LANES + lanes

    def partner(v, d):
        if d < LANES:
            m = (lanes & d) == 0
            return jnp.where(m, pltpu.roll(v, LANES - d, 1),
                             pltpu.roll(v, d, 1)), m
        r = d // LANES
        m = (rows & r) == 0
        return jnp.where(m, pltpu.roll(v, ROWS - r, 0),
                         pltpu.roll(v, r, 0)), m

    kblock = 2
    while kblock < T:
        d = kblock // 2
        while d >= 1:
            sp, low = partner(s2, d)
            ip, _ = partner(i2, d)
            bfr = (s2 > sp) | ((s2 == sp) & (i2 < ip))
            keep = bfr ^ (~low) ^ (((rows * LANES + lanes) & kblock) != 0)
            s2 = jnp.where(keep, s2, sp)
            i2 = jnp.where(keep, i2, ip)
            d //= 2
        kblock *= 2

    sp, low = partner(s2, T // 2)
    ip, _ = partner(i2, T // 2)
    bfr = (s2 > sp) | ((s2 == sp) & (i2 < ip))
    keep = bfr ^ (~low)
    s2 = jnp.where(keep, s2, sp)[:KROWS]
    i2 = jnp.where(keep, i2, ip)[:KROWS]
    hrows = rows[:KROWS]
    hlanes = lanes[:KROWS]

    def hpartner(v, d):
        if d < LANES:
            m = (hlanes & d) == 0
            return jnp.where(m, pltpu.roll(v, LANES - d, 1),
                             pltpu.roll(v, d, 1)), m
        r = d // LANES
        m = (hrows & r) == 0
        return jnp.where(m, pltpu.roll(v, KROWS - r, 0),
                         pltpu.roll(v, r, 0)), m

    d = T // 4
    while d >= 1:
        sp, low = hpartner(s2, d)
        ip, _ = hpartner(i2, d)
        bfr = (s2 > sp) | ((s2 == sp) & (i2 < ip))
        keep = bfr ^ (~low)
        s2 = jnp.where(keep, s2, sp)
        i2 = jnp.where(keep, i2, ip)
        d //= 2

    i_ref[0] = i2
    f_ref[0] = i2 + b * T


def _topk(scores3, b0, nb):
    return pl.pallas_call(
        functools.partial(_topk_kernel, b0),
        grid=(nb,),
        in_specs=[pl.BlockSpec((1, ROWS, LANES), lambda i, b0=b0: (b0 + i, 0, 0))],
        out_specs=[
            pl.BlockSpec((1, KROWS, LANES), lambda i: (i, 0, 0)),
            pl.BlockSpec((1, KROWS, LANES), lambda i: (i, 0, 0)),
        ],
        out_shape=[
            jax.ShapeDtypeStruct((nb, KROWS, LANES), jnp.int32),
            jax.ShapeDtypeStruct((nb, KROWS, LANES), jnp.int32),
        ],
    )(scores3)


_NC, _NS = 2, 16
_NW = _NC * _NS
_RPW = K // _NW
_CH = 16


def _ring(x_hbm, out_ref, idx_v, bufs, gsem, wsem, out_base, nchunk):
    nbuf = len(bufs)

    def start_gather(c):
        return pltpu.async_copy(x_hbm.at[idx_v.at[pl.ds(c * _CH, _CH)]],
                                bufs[c % nbuf], gsem)

    def start_write(c):
        return pltpu.async_copy(bufs[c % nbuf],
                                out_ref.at[pl.ds(out_base + c * _CH, _CH)],
                                wsem)

    g = [None] * nchunk
    w = [None] * nchunk
    waited = set()
    for c in range(min(nbuf - 1, nchunk)):
        g[c] = start_gather(c)
    for c in range(nchunk):
        if c + nbuf - 1 < nchunk:
            if c - 1 >= 0:
                w[c - 1].wait()
                waited.add(c - 1)
            g[c + nbuf - 1] = start_gather(c + nbuf - 1)
        g[c].wait()
        w[c] = start_write(c)
    for c in range(nchunk):
        if c not in waited:
            w[c].wait()


def _gather0_body(idx_hbm, x_hbm, out_ref, idx_v, bufs, gsem, wsem):
    wid = lax.axis_index("s") * _NC + lax.axis_index("c")
    pltpu.sync_copy(
        idx_hbm.at[wid // 2, pl.ds((wid % 2) * _RPW, _RPW)], idx_v)
    _ring(x_hbm, out_ref, idx_v, bufs, gsem, wsem,
          wid * _RPW, _RPW // _CH)


def _gather0(idx2d, x2d, out_ref):
    mesh = plsc.VectorSubcoreMesh(core_axis_name="c", subcore_axis_name="s")
    f = pl.kernel(
        _gather0_body,
        out_type=(),
        mesh=mesh,
        scratch_types=[
            pltpu.VMEM((_RPW,), jnp.int32),
            [pltpu.VMEM((_CH, D), jnp.float32) for _ in range(3)],
            pltpu.SemaphoreType.DMA,
            pltpu.SemaphoreType.DMA,
        ],
    )
    f(idx2d, x2d, out_ref)


def _gatherN_body(nb, row0, idx_hbm, x_hbm, out_ref, idx_v, bufs,
                  gsem, wsem):
    rpw = nb * K // _NW
    wid = lax.axis_index("s") * _NC + lax.axis_index("c")
    pltpu.sync_copy(idx_hbm.at[pl.ds(wid * rpw, rpw)], idx_v)
    _ring(x_hbm, out_ref, idx_v, bufs, gsem, wsem,
          row0 + wid * rpw, rpw // _CH)


def _gatherN(idx1d, x2d, out_ref, nb, row0):
    mesh = plsc.VectorSubcoreMesh(core_axis_name="c", subcore_axis_name="s")
    f = pl.kernel(
        functools.partial(_gatherN_body, nb, row0),
        out_type=(),
        mesh=mesh,
        scratch_types=[
            pltpu.VMEM((nb * K // _NW,), jnp.int32),
            [pltpu.VMEM((_CH, D), jnp.float32) for _ in range(3)],
            pltpu.SemaphoreType.DMA,
            pltpu.SemaphoreType.DMA,
        ],
    )
    f(idx1d, x2d, out_ref)


def kernel(x, W):
    x2d = x.reshape(B * T, D)
    w2d = W.reshape(1, D)
    sel_ref = jax.new_ref(lax.empty((B * K, D), jnp.float32))
    scores_all = _scores(x2d, w2d, 0, B)
    scores3 = scores_all.reshape(B, ROWS, LANES)
    idx0, flat0 = _topk(scores3, 0, 1)
    _gather0(flat0.reshape(KROWS, LANES), x2d, sel_ref)
    idx123, flat123 = _topk(scores3, 1, B - 1)
    _gatherN(flat123.reshape((B - 1) * K), x2d, sel_ref, B - 1, K)
    scores = scores_all.reshape(B, T)
    indices = jnp.concatenate([idx0, idx123]).reshape(B, K)
    selected = jax.freeze(sel_ref).reshape(B, K, D)
    return (selected, indices, scores)


# speedup vs baseline: 1.0153x; 1.0153x over previous
import functools
import jax
import jax.numpy as jnp
from jax import lax
from jax.experimental import pallas as pl
from jax.experimental.pallas import tpu as pltpu
from jax.experimental.pallas import tpu_sc as plsc

B, T, D = 4, 4096, 2048
K = T // 2
ROWS, LANES = 32, 128
KROWS = K // LANES


_BT = 1024
_NSTEP = T // _BT


def _scores_kernel(x_ref, w_ref, o_ref):
    o_ref[0] = lax.dot_general(
        w_ref[...], x_ref[...], (((1,), (1,)), ((), ())),
        preferred_element_type=jnp.float32)


def _scores(x2d, w2d, b0, nb):
    return pl.pallas_call(
        _scores_kernel,
        grid=(nb * _NSTEP,),
        in_specs=[
            pl.BlockSpec((_BT, D), lambda i, b0=b0: (b0 * _NSTEP + i, 0)),
            pl.BlockSpec((1, D), lambda i: (0, 0)),
        ],
        out_specs=pl.BlockSpec((1, 1, _BT), lambda i: (i, 0, 0)),
        out_shape=jax.ShapeDtypeStruct((nb * _NSTEP, 1, _BT), jnp.float32),
    )(x2d, w2d)


def _topk_kernel(b0, s_ref, i_ref, f_ref):
    b = pl.program_id(0) + b0
    s2 = s_ref[0]
    rows = lax.broadcasted_iota(jnp.int32, (ROWS, LANES), 0)
    lanes = lax.broadcasted_iota(jnp.int32, (ROWS, LANES), 1)
    i2 = rows * LANES + lanes

    def partner(v, d):
        if d < LANES:
            m = (lanes & d) == 0
            return jnp.where(m, pltpu.roll(v, LANES - d, 1),
                             pltpu.roll(v, d, 1)), m
        r = d // LANES
        m = (rows & r) == 0
        return jnp.where(m, pltpu.roll(v, ROWS - r, 0),
                         pltpu.roll(v, r, 0)), m

    kblock = 2
    while kblock < T:
        d = kblock // 2
        while d >= 1:
            sp, low = partner(s2, d)
            ip, _ = partner(i2, d)
            bfr = (s2 > sp) | ((s2 == sp) & (i2 < ip))
            keep = bfr ^ (~low) ^ (((rows * LANES + lanes) & kblock) != 0)
            s2 = jnp.where(keep, s2, sp)
            i2 = jnp.where(keep, i2, ip)
            d //= 2
        kblock *= 2

    sp, low = partner(s2, T // 2)
    ip, _ = partner(i2, T // 2)
    bfr = (s2 > sp) | ((s2 == sp) & (i2 < ip))
    keep = bfr ^ (~low)
    s2 = jnp.where(keep, s2, sp)[:KROWS]
    i2 = jnp.where(keep, i2, ip)[:KROWS]
    hrows = rows[:KROWS]
    hlanes = lanes[:KROWS]

    def hpartner(v, d):
        if d < LANES:
            m = (hlanes & d) == 0
            return jnp.where(m, pltpu.roll(v, LANES - d, 1),
                             pltpu.roll(v, d, 1)), m
        r = d // LANES
        m = (hrows & r) == 0
        return jnp.where(m, pltpu.roll(v, KROWS - r, 0),
                         pltpu.roll(v, r, 0)), m

    d = T // 4
    while d >= 1:
        sp, low = hpartner(s2, d)
        ip, _ = hpartner(i2, d)
        bfr = (s2 > sp) | ((s2 == sp) & (i2 < ip))
        keep = bfr ^ (~low)
        s2 = jnp.where(keep, s2, sp)
        i2 = jnp.where(keep, i2, ip)
        d //= 2

    i_ref[0] = i2
    f_ref[0] = i2 + b * T


def _topk(scores3, b0, nb):
    return pl.pallas_call(
        functools.partial(_topk_kernel, b0),
        grid=(nb,),
        in_specs=[pl.BlockSpec((1, ROWS, LANES), lambda i, b0=b0: (b0 + i, 0, 0))],
        out_specs=[
            pl.BlockSpec((1, KROWS, LANES), lambda i: (i, 0, 0)),
            pl.BlockSpec((1, KROWS, LANES), lambda i: (i, 0, 0)),
        ],
        out_shape=[
            jax.ShapeDtypeStruct((nb, KROWS, LANES), jnp.int32),
            jax.ShapeDtypeStruct((nb, KROWS, LANES), jnp.int32),
        ],
    )(scores3)


_NC, _NS = 2, 16
_NW = _NC * _NS
_RPW = K // _NW
_CH = 16


def _ring(x_hbm, out_ref, idx_v, bufs, gsem, wsem, out_base, nchunk):
    nbuf = len(bufs)

    def start_gather(c):
        return pltpu.async_copy(x_hbm.at[idx_v.at[pl.ds(c * _CH, _CH)]],
                                bufs[c % nbuf], gsem)

    def start_write(c):
        return pltpu.async_copy(bufs[c % nbuf],
                                out_ref.at[pl.ds(out_base + c * _CH, _CH)],
                                wsem)

    g = [None] * nchunk
    w = [None] * nchunk
    waited = set()
    for c in range(min(nbuf - 1, nchunk)):
        g[c] = start_gather(c)
    for c in range(nchunk):
        if c + nbuf - 1 < nchunk:
            if c - 1 >= 0:
                w[c - 1].wait()
                waited.add(c - 1)
            g[c + nbuf - 1] = start_gather(c + nbuf - 1)
        g[c].wait()
        w[c] = start_write(c)
    for c in range(nchunk):
        if c not in waited:
            w[c].wait()


def _gather0_body(idx_hbm, x_hbm, out_ref, idx_v, bufs, gsem, wsem):
    wid = lax.axis_index("s") * _NC + lax.axis_index("c")
    pltpu.sync_copy(
        idx_hbm.at[wid // 2, pl.ds((wid % 2) * _RPW, _RPW)], idx_v)
    _ring(x_hbm, out_ref, idx_v, bufs, gsem, wsem,
          wid * _RPW, _RPW // _CH)


def _gather0(idx2d, x2d, out_ref):
    mesh = plsc.VectorSubcoreMesh(core_axis_name="c", subcore_axis_name="s")
    f = pl.kernel(
        _gather0_body,
        out_type=(),
        mesh=mesh,
        scratch_types=[
            pltpu.VMEM((_RPW,), jnp.int32),
            [pltpu.VMEM((_CH, D), jnp.float32) for _ in range(3)],
            pltpu.SemaphoreType.DMA,
            pltpu.SemaphoreType.DMA,
        ],
    )
    f(idx2d, x2d, out_ref)


def _gatherN_body(nb, row0, idx_hbm, x_hbm, out_ref, idx_v, bufs,
                  gsem, wsem):
    rpw = nb * K // _NW
    wid = lax.axis_index("s") * _NC + lax.axis_index("c")
    pltpu.sync_copy(idx_hbm.at[pl.ds(wid * rpw, rpw)], idx_v)
    _ring(x_hbm, out_ref, idx_v, bufs, gsem, wsem,
          row0 + wid * rpw, rpw // _CH)


def _gatherN(idx1d, x2d, out_ref, nb, row0):
    mesh = plsc.VectorSubcoreMesh(core_axis_name="c", subcore_axis_name="s")
    f = pl.kernel(
        functools.partial(_gatherN_body, nb, row0),
        out_type=(),
        mesh=mesh,
        scratch_types=[
            pltpu.VMEM((nb * K // _NW,), jnp.int32),
            [pltpu.VMEM((_CH, D), jnp.float32) for _ in range(3)],
            pltpu.SemaphoreType.DMA,
            pltpu.SemaphoreType.DMA,
        ],
    )
    f(idx1d, x2d, out_ref)


def kernel(x, W):
    x2d = x.reshape(B * T, D)
    w2d = W.reshape(1, D)
    sel_ref = jax.new_ref(lax.empty((B * K, D), jnp.float32))
    scores_all = _scores(x2d, w2d, 0, B)
    scores3 = scores_all.reshape(B, ROWS, LANES)
    idx0, flat0 = _topk(scores3, 0, 1)
    _gather0(flat0.reshape(KROWS, LANES), x2d, sel_ref)
    idx123, flat123 = _topk(scores3, 1, B - 1)
    _gatherN(flat123.reshape((B - 1) * K), x2d, sel_ref, B - 1, K)
    scores = scores_all.reshape(B, T)
    indices = jnp.concatenate([idx0, idx123]).reshape(B, K)
    selected = jax.freeze(sel_ref).reshape(B, K, D)
    return (selected, indices, scores)
